# baseline (speedup 1.0000x reference)
import jax
import jax.numpy as jnp
from jax.experimental import pallas as pl
from jax.experimental.pallas import tpu as pltpu

_HIGH = jax.lax.Precision.HIGHEST
_MATCH = jax.lax.Precision.DEFAULT
_TILE = 2048


def _dotT(a, b):
    return jax.lax.dot_general(a, b, (((1,), (1,)), ((), ())),
                               precision=_MATCH)


def _pass1_body(n_slots, q_ref, w_ref, b_ref, k_ref, occ_ref,
                amps_ref, m_ref, s_ref, qq_ref):
    i = pl.program_id(0)
    tile = k_ref.shape[0]
    bsz = q_ref.shape[0]

    @pl.when(i == 0)
    def _init():
        kd2 = w_ref.shape[0]
        q_enc = _dotT(q_ref[...], w_ref[...]) + b_ref[...]
        r = jax.lax.broadcasted_iota(jnp.int32, (kd2, kd2), 0)
        c = jax.lax.broadcasted_iota(jnp.int32, (kd2, kd2), 1)
        half = c // 2
        even = (c % 2) == 0
        kd = kd2 // 2
        one = jnp.float32(1.0)
        zero = jnp.float32(0.0)
        pa = (jnp.where(even & (r == half), one, zero)
              + jnp.where((~even) & (r == kd + half), one, zero))
        pb = (jnp.where(even & (r == kd + half), -one, zero)
              + jnp.where((~even) & (r == half), one, zero))
        qa = jax.lax.dot_general(q_enc, pa, (((1,), (0,)), ((), ())),
                                 precision=_HIGH)
        qb = jax.lax.dot_general(q_enc, pb, (((1,), (0,)), ((), ())),
                                 precision=_HIGH)
        qq_ref[...] = jnp.concatenate([qa, qb], axis=0)
        m_ref[...] = jnp.full((bsz, 1), -jnp.inf, dtype=jnp.float32)
        s_ref[...] = jnp.zeros((bsz, 1), dtype=jnp.float32)

    inner = _dotT(qq_ref[...], k_ref[...])
    ir = inner[:bsz, :]
    ii = inner[bsz:, :]
    amp = (ir * ir + ii * ii) * occ_ref[...]
    col = i * tile + jax.lax.broadcasted_iota(jnp.int32, (1, tile), 1)
    amp = jnp.where(col < n_slots, amp, -jnp.inf)
    amps_ref[...] = amp

    m_old = m_ref[...]
    m_new = jnp.maximum(m_old, jnp.max(amp, axis=1, keepdims=True))
    p = jnp.exp(amp - m_new)
    s_ref[...] = (s_ref[...] * jnp.exp(m_old - m_new)
                  + jnp.sum(p, axis=1, keepdims=True))
    m_ref[...] = m_new


def _pass2_body(n_slots, amps_ref, c_ref, m_ref, s_ref, attn_ref, cont_ref):
    i = pl.program_id(0)
    tile = c_ref.shape[0]
    col = i * tile + jax.lax.broadcasted_iota(jnp.int32, (1, tile), 1)
    valid = col < n_slots
    p = jnp.exp(amps_ref[...] - m_ref[...]) / s_ref[...]
    p = jnp.where(valid, p, 0.0)
    attn_ref[...] = p
    row_valid = (i * tile
                 + jax.lax.broadcasted_iota(jnp.int32, (tile, 1), 0)) < n_slots
    c_blk = jnp.where(row_valid, c_ref[...], 0.0)
    acc = jax.lax.dot_general(p, c_blk, (((1,), (0,)), ((), ())),
                              precision=_MATCH)

    @pl.when(i == 0)
    def _first():
        cont_ref[...] = acc

    @pl.when(i != 0)
    def _rest():
        cont_ref[...] += acc


def kernel(query, contents, mem_keys, occupancy, W, b):
    n_slots, key_dim, _ = mem_keys.shape
    mem_dim = contents.shape[1]
    bsz = query.shape[0]

    k_flat = mem_keys.reshape(n_slots, 2 * key_dim)
    c_flat = contents.reshape(n_slots, 2 * mem_dim)
    occ2 = occupancy.reshape(1, n_slots)
    b2 = b.reshape(1, 2 * key_dim)

    tile = _TILE
    nb = pl.cdiv(n_slots, tile)

    import functools as _ft

    amps, m, s = pl.pallas_call(
        _ft.partial(_pass1_body, n_slots),
        grid=(nb,),
        in_specs=[
            pl.BlockSpec((bsz, key_dim), lambda i: (0, 0)),
            pl.BlockSpec((2 * key_dim, key_dim), lambda i: (0, 0)),
            pl.BlockSpec((1, 2 * key_dim), lambda i: (0, 0)),
            pl.BlockSpec((tile, 2 * key_dim), lambda i: (i, 0)),
            pl.BlockSpec((1, tile), lambda i: (0, i)),
        ],
        out_specs=[
            pl.BlockSpec((bsz, tile), lambda i: (0, i)),
            pl.BlockSpec((bsz, 1), lambda i: (0, 0)),
            pl.BlockSpec((bsz, 1), lambda i: (0, 0)),
        ],
        out_shape=[
            jax.ShapeDtypeStruct((bsz, n_slots), jnp.float32),
            jax.ShapeDtypeStruct((bsz, 1), jnp.float32),
            jax.ShapeDtypeStruct((bsz, 1), jnp.float32),
        ],
        scratch_shapes=[pltpu.VMEM((2 * bsz, 2 * key_dim), jnp.float32)],
    )(query, W, b2, k_flat, occ2)

    attn, cont = pl.pallas_call(
        _ft.partial(_pass2_body, n_slots),
        grid=(nb,),
        in_specs=[
            pl.BlockSpec((bsz, tile), lambda i: (0, i)),
            pl.BlockSpec((tile, 2 * mem_dim), lambda i: (i, 0)),
            pl.BlockSpec((bsz, 1), lambda i: (0, 0)),
            pl.BlockSpec((bsz, 1), lambda i: (0, 0)),
        ],
        out_specs=[
            pl.BlockSpec((bsz, tile), lambda i: (0, i)),
            pl.BlockSpec((bsz, 2 * mem_dim), lambda i: (0, 0)),
        ],
        out_shape=[
            jax.ShapeDtypeStruct((bsz, n_slots), jnp.float32),
            jax.ShapeDtypeStruct((bsz, 2 * mem_dim), jnp.float32),
        ],
    )(amps, c_flat, m, s)

    content = cont.reshape(bsz, mem_dim, 2)[..., 0]
    return (content, attn)


# speedup vs baseline: 4.2867x; 4.2867x over previous
import functools

import jax
import jax.numpy as jnp
from jax.experimental import pallas as pl
from jax.experimental.pallas import tpu as pltpu

_MATCH = jax.lax.Precision.DEFAULT
_TILE = 2048


def _pass1_body(n_slots, q_ref, w_ref, b_ref, k_ref, occ_ref,
                amps_ref, m_ref, s_ref, qq_ref):
    i = pl.program_id(0)
    tile = k_ref.shape[2]
    bsz = q_ref.shape[0]

    @pl.when(i == 0)
    def _init():
        q_enc = jax.lax.dot_general(q_ref[...], w_ref[...],
                                    (((1,), (1,)), ((), ())),
                                    precision=_MATCH) + b_ref[...]
        kd = w_ref.shape[0] // 2
        q_real = q_enc[:, :kd]
        q_imag = q_enc[:, kd:]
        qa = jnp.concatenate([q_real, -q_imag], axis=0)
        qb = jnp.concatenate([q_imag, q_real], axis=0)
        qq_ref[...] = jnp.concatenate([qa, qb], axis=0)
        m_ref[...] = jnp.full((bsz, 1), -jnp.inf, dtype=jnp.float32)
        s_ref[...] = jnp.zeros((bsz, 1), dtype=jnp.float32)

    k_real = k_ref[:, 0, :]
    k_imag = k_ref[:, 1, :]
    qa = qq_ref[:2 * bsz, :]
    qb = qq_ref[2 * bsz:, :]
    inner = (jax.lax.dot_general(qa, k_real, (((1,), (0,)), ((), ())),
                                 precision=_MATCH)
             + jax.lax.dot_general(qb, k_imag, (((1,), (0,)), ((), ())),
                                   precision=_MATCH))
    ir = inner[:bsz, :]
    ii = inner[bsz:, :]
    amp = (ir * ir + ii * ii) * occ_ref[...]
    col = i * tile + jax.lax.broadcasted_iota(jnp.int32, (1, tile), 1)
    amp = jnp.where(col < n_slots, amp, -jnp.inf)
    amps_ref[...] = amp

    m_old = m_ref[...]
    m_new = jnp.maximum(m_old, jnp.max(amp, axis=1, keepdims=True))
    p = jnp.exp(amp - m_new)
    s_ref[...] = (s_ref[...] * jnp.exp(m_old - m_new)
                  + jnp.sum(p, axis=1, keepdims=True))
    m_ref[...] = m_new


def _pass2_body(n_slots, amps_ref, c_ref, m_ref, s_ref, attn_ref, cont_ref):
    i = pl.program_id(0)
    tile = c_ref.shape[0]
    col = i * tile + jax.lax.broadcasted_iota(jnp.int32, (1, tile), 1)
    p = jnp.exp(amps_ref[...] - m_ref[...]) / s_ref[...]
    p = jnp.where(col < n_slots, p, 0.0)
    attn_ref[...] = p
    row_valid = (i * tile
                 + jax.lax.broadcasted_iota(jnp.int32, (tile, 1), 0)) < n_slots
    c_blk = jnp.where(row_valid, c_ref[:, 0, :], 0.0)
    acc = jax.lax.dot_general(p, c_blk, (((1,), (0,)), ((), ())),
                              precision=_MATCH)

    @pl.when(i == 0)
    def _first():
        cont_ref[...] = acc

    @pl.when(i != 0)
    def _rest():
        cont_ref[...] += acc


def kernel(query, contents, mem_keys, occupancy, W, b):
    n_slots, key_dim, _ = mem_keys.shape
    mem_dim = contents.shape[1]
    bsz = query.shape[0]

    k_t = mem_keys.transpose(1, 2, 0)
    c_t = contents.transpose(0, 2, 1)
    occ2 = occupancy.reshape(1, n_slots)
    b2 = b.reshape(1, 2 * key_dim)

    tile = _TILE
    nb = pl.cdiv(n_slots, tile)

    amps, m, s = pl.pallas_call(
        functools.partial(_pass1_body, n_slots),
        grid=(nb,),
        in_specs=[
            pl.BlockSpec((bsz, key_dim), lambda i: (0, 0)),
            pl.BlockSpec((2 * key_dim, key_dim), lambda i: (0, 0)),
            pl.BlockSpec((1, 2 * key_dim), lambda i: (0, 0)),
            pl.BlockSpec((key_dim, 2, tile), lambda i: (0, 0, i)),
            pl.BlockSpec((1, tile), lambda i: (0, i)),
        ],
        out_specs=[
            pl.BlockSpec((bsz, tile), lambda i: (0, i)),
            pl.BlockSpec((bsz, 1), lambda i: (0, 0)),
            pl.BlockSpec((bsz, 1), lambda i: (0, 0)),
        ],
        out_shape=[
            jax.ShapeDtypeStruct((bsz, n_slots), jnp.float32),
            jax.ShapeDtypeStruct((bsz, 1), jnp.float32),
            jax.ShapeDtypeStruct((bsz, 1), jnp.float32),
        ],
        scratch_shapes=[pltpu.VMEM((4 * bsz, key_dim), jnp.float32)],
    )(query, W, b2, k_t, occ2)

    attn, cont = pl.pallas_call(
        functools.partial(_pass2_body, n_slots),
        grid=(nb,),
        in_specs=[
            pl.BlockSpec((bsz, tile), lambda i: (0, i)),
            pl.BlockSpec((tile, 2, mem_dim), lambda i: (i, 0, 0)),
            pl.BlockSpec((bsz, 1), lambda i: (0, 0)),
            pl.BlockSpec((bsz, 1), lambda i: (0, 0)),
        ],
        out_specs=[
            pl.BlockSpec((bsz, tile), lambda i: (0, i)),
            pl.BlockSpec((bsz, mem_dim), lambda i: (0, 0)),
        ],
        out_shape=[
            jax.ShapeDtypeStruct((bsz, n_slots), jnp.float32),
            jax.ShapeDtypeStruct((bsz, mem_dim), jnp.float32),
        ],
    )(amps, c_t, m, s)

    return (cont, attn)


# speedup vs baseline: 5.0833x; 1.1858x over previous
import functools

import jax
import jax.numpy as jnp
from jax.experimental import pallas as pl
from jax.experimental.pallas import tpu as pltpu

_MATCH = jax.lax.Precision.DEFAULT
_TILE = 2048


def _pass1_body(n_slots, tile, nb, q_ref, w_ref, b_ref, occ_ref, k_ref,
                amps_ref, m_ref, s_ref, qq_ref):
    i = pl.program_id(0)
    bsz = q_ref.shape[0]
    kd = w_ref.shape[0] // 2

    @pl.when(i == 0)
    def _prologue():
        q_enc = jax.lax.dot_general(q_ref[...], w_ref[...],
                                    (((1,), (1,)), ((), ())),
                                    precision=_MATCH) + b_ref[...]
        q_real = q_enc[:, :kd]
        q_imag = q_enc[:, kd:]
        qa = jnp.concatenate([q_real, -q_imag], axis=0)
        qb = jnp.concatenate([q_imag, q_real], axis=0)
        qq_ref[...] = jnp.concatenate([qa, qb], axis=0)
        m_ref[...] = jnp.full((bsz, 1), -jnp.inf, dtype=jnp.float32)
        s_ref[...] = jnp.zeros((bsz, 1), dtype=jnp.float32)

    k_real = k_ref[:, 0, :]
    k_imag = k_ref[:, 1, :]
    qa = qq_ref[:2 * bsz, :]
    qb = qq_ref[2 * bsz:, :]
    inner = (jax.lax.dot_general(qa, k_real, (((1,), (0,)), ((), ())),
                                 precision=_MATCH)
             + jax.lax.dot_general(qb, k_imag, (((1,), (0,)), ((), ())),
                                   precision=_MATCH))
    ir = inner[:bsz, :]
    ii = inner[bsz:, :]
    amp = (ir * ir + ii * ii) * occ_ref[...]
    col = i * tile + jax.lax.broadcasted_iota(jnp.int32, (1, tile), 1)
    amp = jnp.where(col < n_slots, amp, -jnp.inf)
    amps_ref[...] = amp

    m_old = m_ref[...]
    m_new = jnp.maximum(m_old, jnp.max(amp, axis=1, keepdims=True))
    p = jnp.exp(amp - m_new)
    s_ref[...] = (s_ref[...] * jnp.exp(m_old - m_new)
                  + jnp.sum(p, axis=1, keepdims=True))
    m_ref[...] = m_new


def _pass2_body(n_slots, tile, nb, amps_ref, m_ref, s_ref, c_any,
                attn_ref, cont_ref, cbuf, csem):
    i = pl.program_id(0)
    tail = n_slots - (nb - 1) * tile
    last = nb - 1

    def start_full(idx, slot):
        pltpu.make_async_copy(
            c_any.at[pl.ds(idx * tile, tile), 0],
            cbuf.at[slot],
            csem.at[slot]).start()

    def start_tail(slot):
        pltpu.make_async_copy(
            c_any.at[pl.ds(last * tile, tail), 0],
            cbuf.at[slot, pl.ds(0, tail)],
            csem.at[slot]).start()

    def wait_full(slot):
        pltpu.make_async_copy(
            c_any.at[pl.ds(0, tile), 0],
            cbuf.at[slot],
            csem.at[slot]).wait()

    def wait_tail(slot):
        pltpu.make_async_copy(
            c_any.at[pl.ds(0, tail), 0],
            cbuf.at[slot, pl.ds(0, tail)],
            csem.at[slot]).wait()

    slot = jax.lax.rem(i, 2)
    nxt = jax.lax.rem(i + 1, 2)

    @pl.when(i == 0)
    def _prologue():
        start_full(0, 0)

    @pl.when(i + 1 < last)
    def _prefetch_full():
        start_full(i + 1, nxt)

    @pl.when(i + 1 == last)
    def _prefetch_tail():
        start_tail(nxt)

    col = i * tile + jax.lax.broadcasted_iota(jnp.int32, (1, tile), 1)
    p = jnp.exp(amps_ref[...] - m_ref[...]) / s_ref[...]
    p = jnp.where(col < n_slots, p, 0.0)
    attn_ref[...] = p

    @pl.when(i < last)
    def _wait_f():
        wait_full(slot)

    @pl.when(i == last)
    def _wait_t():
        wait_tail(slot)

    acc = jax.lax.dot_general(p, cbuf[slot], (((1,), (0,)), ((), ())),
                              precision=_MATCH)

    @pl.when(i == 0)
    def _first():
        cont_ref[...] = acc

    @pl.when(i != 0)
    def _rest():
        cont_ref[...] += acc


def kernel(query, contents, mem_keys, occupancy, W, b):
    n_slots, key_dim, _ = mem_keys.shape
    mem_dim = contents.shape[1]
    bsz = query.shape[0]

    k_t = mem_keys.transpose(1, 2, 0)
    c_t = contents.transpose(0, 2, 1)
    occ2 = occupancy.reshape(1, n_slots)
    b2 = b.reshape(1, 2 * key_dim)

    tile = _TILE
    nb = pl.cdiv(n_slots, tile)
    n_pad = nb * tile

    amps, m, s = pl.pallas_call(
        functools.partial(_pass1_body, n_slots, tile, nb),
        grid=(nb,),
        in_specs=[
            pl.BlockSpec((bsz, key_dim), lambda i: (0, 0)),
            pl.BlockSpec((2 * key_dim, key_dim), lambda i: (0, 0)),
            pl.BlockSpec((1, 2 * key_dim), lambda i: (0, 0)),
            pl.BlockSpec((1, tile), lambda i: (0, i)),
            pl.BlockSpec((key_dim, 2, tile), lambda i: (0, 0, i)),
        ],
        out_specs=[
            pl.BlockSpec((bsz, tile), lambda i: (0, i)),
            pl.BlockSpec((bsz, 1), lambda i: (0, 0)),
            pl.BlockSpec((bsz, 1), lambda i: (0, 0)),
        ],
        out_shape=[
            jax.ShapeDtypeStruct((bsz, n_pad), jnp.float32),
            jax.ShapeDtypeStruct((bsz, 1), jnp.float32),
            jax.ShapeDtypeStruct((bsz, 1), jnp.float32),
        ],
        scratch_shapes=[
            pltpu.VMEM((4 * bsz, key_dim), jnp.float32),
        ],
    )(query, W, b2, occ2, k_t)

    attn, cont = pl.pallas_call(
        functools.partial(_pass2_body, n_slots, tile, nb),
        grid=(nb,),
        in_specs=[
            pl.BlockSpec((bsz, tile), lambda i: (0, i)),
            pl.BlockSpec((bsz, 1), lambda i: (0, 0)),
            pl.BlockSpec((bsz, 1), lambda i: (0, 0)),
            pl.BlockSpec(memory_space=pl.ANY),
        ],
        out_specs=[
            pl.BlockSpec((bsz, tile), lambda i: (0, i)),
            pl.BlockSpec((bsz, mem_dim), lambda i: (0, 0)),
        ],
        out_shape=[
            jax.ShapeDtypeStruct((bsz, n_slots), jnp.float32),
            jax.ShapeDtypeStruct((bsz, mem_dim), jnp.float32),
        ],
        scratch_shapes=[
            pltpu.VMEM((2, tile, mem_dim), jnp.float32),
            pltpu.SemaphoreType.DMA((2,)),
        ],
    )(amps, m, s, c_t)

    return (cont, attn)


# speedup vs baseline: 5.7158x; 1.1244x over previous
import functools

import jax
import jax.numpy as jnp
from jax.experimental import pallas as pl
from jax.experimental.pallas import tpu as pltpu

_MATCH = jax.lax.Precision.DEFAULT
_TILE = 2048


def _body(n_slots, tile, nb, q_ref, w_ref, b_ref, occ_ref, k_ref, c_any,
          attn_ref, cont_ref, qq_ref, amps_scr, m_ref, s_ref, cbuf, csem):
    phase = pl.program_id(0)
    i = pl.program_id(1)
    bsz = q_ref.shape[0]
    kd = w_ref.shape[0] // 2
    tail = n_slots - (nb - 1) * tile
    last = nb - 1

    def c_start_full(idx, slot):
        pltpu.make_async_copy(
            c_any.at[pl.ds(idx * tile, tile), 0],
            cbuf.at[slot],
            csem.at[slot]).start()

    def c_start_tail(slot):
        pltpu.make_async_copy(
            c_any.at[pl.ds(last * tile, tail), 0],
            cbuf.at[slot, pl.ds(0, tail)],
            csem.at[slot]).start()

    def c_wait_full(slot):
        pltpu.make_async_copy(
            c_any.at[pl.ds(0, tile), 0],
            cbuf.at[slot],
            csem.at[slot]).wait()

    def c_wait_tail(slot):
        pltpu.make_async_copy(
            c_any.at[pl.ds(0, tail), 0],
            cbuf.at[slot, pl.ds(0, tail)],
            csem.at[slot]).wait()

    @pl.when((phase == 0) & (i == 0))
    def _prologue():
        q_enc = jax.lax.dot_general(q_ref[...], w_ref[...],
                                    (((1,), (1,)), ((), ())),
                                    precision=_MATCH) + b_ref[...]
        q_real = q_enc[:, :kd]
        q_imag = q_enc[:, kd:]
        qa = jnp.concatenate([q_real, -q_imag], axis=0)
        qb = jnp.concatenate([q_imag, q_real], axis=0)
        qq_ref[...] = jnp.concatenate([qa, qb], axis=0)
        m_ref[...] = jnp.full((bsz, 1), -jnp.inf, dtype=jnp.float32)
        s_ref[...] = jnp.zeros((bsz, 1), dtype=jnp.float32)

    @pl.when(phase == 0)
    def _phase0():
        k_real = k_ref[:, 0, :]
        k_imag = k_ref[:, 1, :]
        qa = qq_ref[:2 * bsz, :]
        qb = qq_ref[2 * bsz:, :]
        inner = (jax.lax.dot_general(qa, k_real, (((1,), (0,)), ((), ())),
                                     precision=_MATCH)
                 + jax.lax.dot_general(qb, k_imag, (((1,), (0,)), ((), ())),
                                       precision=_MATCH))
        ir = inner[:bsz, :]
        ii = inner[bsz:, :]
        amp = (ir * ir + ii * ii) * occ_ref[...]
        col = i * tile + jax.lax.broadcasted_iota(jnp.int32, (1, tile), 1)
        amp = jnp.where(col < n_slots, amp, -jnp.inf)
        amps_scr[:, pl.ds(i * tile, tile)] = amp

        m_old = m_ref[...]
        m_new = jnp.maximum(m_old, jnp.max(amp, axis=1, keepdims=True))
        p = jnp.exp(amp - m_new)
        s_ref[...] = (s_ref[...] * jnp.exp(m_old - m_new)
                      + jnp.sum(p, axis=1, keepdims=True))
        m_ref[...] = m_new

        @pl.when(i == last)
        def _prefetch_first_contents():
            c_start_full(0, 0)

    @pl.when(phase == 1)
    def _phase1():
        slot = jax.lax.rem(i, 2)
        nxt = jax.lax.rem(i + 1, 2)

        @pl.when(i + 1 < last)
        def _prefetch_full():
            c_start_full(i + 1, nxt)

        @pl.when(i + 1 == last)
        def _prefetch_tail():
            c_start_tail(nxt)

        col = i * tile + jax.lax.broadcasted_iota(jnp.int32, (1, tile), 1)
        amp = amps_scr[:, pl.ds(i * tile, tile)]
        p = jnp.exp(amp - m_ref[...]) / s_ref[...]
        p = jnp.where(col < n_slots, p, 0.0)
        attn_ref[...] = p

        @pl.when(i < last)
        def _wait_f():
            c_wait_full(slot)

        @pl.when(i == last)
        def _wait_t():
            c_wait_tail(slot)

        acc = jax.lax.dot_general(p, cbuf[slot], (((1,), (0,)), ((), ())),
                                  precision=_MATCH)

        @pl.when(i == 0)
        def _first():
            cont_ref[...] = acc

        @pl.when(i != 0)
        def _rest():
            cont_ref[...] += acc


def kernel(query, contents, mem_keys, occupancy, W, b):
    n_slots, key_dim, _ = mem_keys.shape
    mem_dim = contents.shape[1]
    bsz = query.shape[0]

    k_t = mem_keys.transpose(1, 2, 0)
    c_t = contents.transpose(0, 2, 1)
    occ2 = occupancy.reshape(1, n_slots)
    b2 = b.reshape(1, 2 * key_dim)

    tile = _TILE
    nb = pl.cdiv(n_slots, tile)
    n_pad = nb * tile

    attn, cont = pl.pallas_call(
        functools.partial(_body, n_slots, tile, nb),
        grid=(2, nb),
        in_specs=[
            pl.BlockSpec((bsz, key_dim), lambda p, i: (0, 0)),
            pl.BlockSpec((2 * key_dim, key_dim), lambda p, i: (0, 0)),
            pl.BlockSpec((1, 2 * key_dim), lambda p, i: (0, 0)),
            pl.BlockSpec((1, tile), lambda p, i: (0, i * (1 - p))),
            pl.BlockSpec((key_dim, 2, tile), lambda p, i: (0, 0, i * (1 - p))),
            pl.BlockSpec(memory_space=pl.ANY),
        ],
        out_specs=[
            pl.BlockSpec((bsz, tile), lambda p, i: (0, i * p)),
            pl.BlockSpec((bsz, mem_dim), lambda p, i: (0, 0)),
        ],
        out_shape=[
            jax.ShapeDtypeStruct((bsz, n_slots), jnp.float32),
            jax.ShapeDtypeStruct((bsz, mem_dim), jnp.float32),
        ],
        scratch_shapes=[
            pltpu.VMEM((4 * bsz, key_dim), jnp.float32),
            pltpu.VMEM((bsz, n_pad), jnp.float32),
            pltpu.VMEM((bsz, 1), jnp.float32),
            pltpu.VMEM((bsz, 1), jnp.float32),
            pltpu.VMEM((2, tile, mem_dim), jnp.float32),
            pltpu.SemaphoreType.DMA((2,)),
        ],
        compiler_params=pltpu.CompilerParams(
            vmem_limit_bytes=100 * 1024 * 1024,
        ),
    )(query, W, b2, occ2, k_t, c_t)

    return (cont, attn)


# speedup vs baseline: 7.8420x; 1.3720x over previous
import functools

import jax
import jax.numpy as jnp
from jax.experimental import pallas as pl
from jax.experimental.pallas import tpu as pltpu

_MATCH = jax.lax.Precision.DEFAULT
_TILE = 4096


def _body(n_slots, tile, nb, q_ref, w_ref, b_ref, occ_ref, k_ref, c_any,
          attn_ref, cont_ref, qq_ref, amps_scr, m_ref, s_ref, cbuf, csem):
    phase = pl.program_id(0)
    i = pl.program_id(1)
    bsz = q_ref.shape[0]
    kd = w_ref.shape[0] // 2
    tail = n_slots - (nb - 1) * tile
    last = nb - 1

    def c_start_full(idx, slot):
        pltpu.make_async_copy(
            c_any.at[pl.ds(idx * tile, tile), 0],
            cbuf.at[slot],
            csem.at[slot]).start()

    def c_start_tail(slot):
        pltpu.make_async_copy(
            c_any.at[pl.ds(last * tile, tail), 0],
            cbuf.at[slot, pl.ds(0, tail)],
            csem.at[slot]).start()

    def c_wait_full(slot):
        pltpu.make_async_copy(
            c_any.at[pl.ds(0, tile), 0],
            cbuf.at[slot],
            csem.at[slot]).wait()

    def c_wait_tail(slot):
        pltpu.make_async_copy(
            c_any.at[pl.ds(0, tail), 0],
            cbuf.at[slot, pl.ds(0, tail)],
            csem.at[slot]).wait()

    @pl.when((phase == 0) & (i == 0))
    def _prologue():
        q_enc = jax.lax.dot_general(q_ref[...], w_ref[...],
                                    (((1,), (1,)), ((), ())),
                                    precision=_MATCH) + b_ref[...]
        q_real = q_enc[:, :kd]
        q_imag = q_enc[:, kd:]
        qa = jnp.concatenate([q_real, -q_imag], axis=0)
        qb = jnp.concatenate([q_imag, q_real], axis=0)
        qq_ref[...] = jnp.concatenate([qa, qb], axis=0)
        m_ref[...] = jnp.full((bsz, 1), -jnp.inf, dtype=jnp.float32)
        s_ref[...] = jnp.zeros((bsz, 1), dtype=jnp.float32)

    @pl.when(phase == 0)
    def _phase0():
        k_real = k_ref[:, 0, :]
        k_imag = k_ref[:, 1, :]
        qa = qq_ref[:2 * bsz, :]
        qb = qq_ref[2 * bsz:, :]
        inner = (jax.lax.dot_general(qa, k_real, (((1,), (0,)), ((), ())),
                                     precision=_MATCH)
                 + jax.lax.dot_general(qb, k_imag, (((1,), (0,)), ((), ())),
                                       precision=_MATCH))
        ir = inner[:bsz, :]
        ii = inner[bsz:, :]
        amp = (ir * ir + ii * ii) * occ_ref[...]
        col = i * tile + jax.lax.broadcasted_iota(jnp.int32, (1, tile), 1)
        amp = jnp.where(col < n_slots, amp, -jnp.inf)
        amps_scr[:, pl.ds(i * tile, tile)] = amp

        m_old = m_ref[...]
        m_new = jnp.maximum(m_old, jnp.max(amp, axis=1, keepdims=True))
        p = jnp.exp(amp - m_new)
        s_ref[...] = (s_ref[...] * jnp.exp(m_old - m_new)
                      + jnp.sum(p, axis=1, keepdims=True))
        m_ref[...] = m_new

        @pl.when(i == last)
        def _prefetch_first_contents():
            c_start_full(0, 0)

    @pl.when(phase == 1)
    def _phase1():
        slot = jax.lax.rem(i, 2)
        nxt = jax.lax.rem(i + 1, 2)

        @pl.when(i + 1 < last)
        def _prefetch_full():
            c_start_full(i + 1, nxt)

        @pl.when(i + 1 == last)
        def _prefetch_tail():
            c_start_tail(nxt)

        col = i * tile + jax.lax.broadcasted_iota(jnp.int32, (1, tile), 1)
        amp = amps_scr[:, pl.ds(i * tile, tile)]
        p = jnp.exp(amp - m_ref[...]) / s_ref[...]
        p = jnp.where(col < n_slots, p, 0.0)
        attn_ref[...] = p

        @pl.when(i < last)
        def _wait_f():
            c_wait_full(slot)

        @pl.when(i == last)
        def _wait_t():
            c_wait_tail(slot)

        acc = jax.lax.dot_general(p, cbuf[slot], (((1,), (0,)), ((), ())),
                                  precision=_MATCH)

        @pl.when(i == 0)
        def _first():
            cont_ref[...] = acc

        @pl.when(i != 0)
        def _rest():
            cont_ref[...] += acc


def kernel(query, contents, mem_keys, occupancy, W, b):
    n_slots, key_dim, _ = mem_keys.shape
    mem_dim = contents.shape[1]
    bsz = query.shape[0]

    k_t = mem_keys.transpose(1, 2, 0)
    c_t = contents.transpose(0, 2, 1)
    occ2 = occupancy.reshape(1, n_slots)
    b2 = b.reshape(1, 2 * key_dim)

    tile = _TILE
    nb = pl.cdiv(n_slots, tile)
    n_pad = nb * tile

    attn, cont = pl.pallas_call(
        functools.partial(_body, n_slots, tile, nb),
        grid=(2, nb),
        in_specs=[
            pl.BlockSpec((bsz, key_dim), lambda p, i: (0, 0)),
            pl.BlockSpec((2 * key_dim, key_dim), lambda p, i: (0, 0)),
            pl.BlockSpec((1, 2 * key_dim), lambda p, i: (0, 0)),
            pl.BlockSpec((1, tile), lambda p, i: (0, i * (1 - p))),
            pl.BlockSpec((key_dim, 2, tile), lambda p, i: (0, 0, i * (1 - p))),
            pl.BlockSpec(memory_space=pl.ANY),
        ],
        out_specs=[
            pl.BlockSpec((bsz, tile), lambda p, i: (0, i * p)),
            pl.BlockSpec((bsz, mem_dim), lambda p, i: (0, 0)),
        ],
        out_shape=[
            jax.ShapeDtypeStruct((bsz, n_slots), jnp.float32),
            jax.ShapeDtypeStruct((bsz, mem_dim), jnp.float32),
        ],
        scratch_shapes=[
            pltpu.VMEM((4 * bsz, key_dim), jnp.float32),
            pltpu.VMEM((bsz, n_pad), jnp.float32),
            pltpu.VMEM((bsz, 1), jnp.float32),
            pltpu.VMEM((bsz, 1), jnp.float32),
            pltpu.VMEM((2, tile, mem_dim), jnp.float32),
            pltpu.SemaphoreType.DMA((2,)),
        ],
        compiler_params=pltpu.CompilerParams(
            vmem_limit_bytes=100 * 1024 * 1024,
        ),
    )(query, W, b2, occ2, k_t, c_t)

    return (cont, attn)


# speedup vs baseline: 9.2090x; 1.1743x over previous
import functools

import jax
import jax.numpy as jnp
from jax.experimental import pallas as pl
from jax.experimental.pallas import tpu as pltpu

_MATCH = jax.lax.Precision.DEFAULT
_TILE = 8192


def _body(n_slots, tile, nb, q_ref, w_ref, b_ref, occ_ref, k_ref, c_any,
          attn_ref, cont_ref, qq_ref, amps_scr, m_ref, s_ref, cbuf, csem):
    phase = pl.program_id(0)
    i = pl.program_id(1)
    bsz = q_ref.shape[0]
    kd = w_ref.shape[0] // 2
    tail = n_slots - (nb - 1) * tile
    last = nb - 1

    def c_start_full(idx, slot):
        pltpu.make_async_copy(
            c_any.at[pl.ds(idx * tile, tile), 0],
            cbuf.at[slot],
            csem.at[slot]).start()

    def c_start_tail(slot):
        pltpu.make_async_copy(
            c_any.at[pl.ds(last * tile, tail), 0],
            cbuf.at[slot, pl.ds(0, tail)],
            csem.at[slot]).start()

    def c_wait_full(slot):
        pltpu.make_async_copy(
            c_any.at[pl.ds(0, tile), 0],
            cbuf.at[slot],
            csem.at[slot]).wait()

    def c_wait_tail(slot):
        pltpu.make_async_copy(
            c_any.at[pl.ds(0, tail), 0],
            cbuf.at[slot, pl.ds(0, tail)],
            csem.at[slot]).wait()

    @pl.when((phase == 0) & (i == 0))
    def _prologue():
        q_enc = jax.lax.dot_general(q_ref[...], w_ref[...],
                                    (((1,), (1,)), ((), ())),
                                    precision=_MATCH) + b_ref[...]
        q_real = q_enc[:, :kd]
        q_imag = q_enc[:, kd:]
        qa = jnp.concatenate([q_real, -q_imag], axis=0)
        qb = jnp.concatenate([q_imag, q_real], axis=0)
        qq_ref[...] = jnp.concatenate([qa, qb], axis=0)
        m_ref[...] = jnp.full((bsz, 1), -jnp.inf, dtype=jnp.float32)
        s_ref[...] = jnp.zeros((bsz, 1), dtype=jnp.float32)

    @pl.when(phase == 0)
    def _phase0():
        k_real = k_ref[:, 0, :]
        k_imag = k_ref[:, 1, :]
        qa = qq_ref[:2 * bsz, :]
        qb = qq_ref[2 * bsz:, :]
        inner = (jax.lax.dot_general(qa, k_real, (((1,), (0,)), ((), ())),
                                     precision=_MATCH)
                 + jax.lax.dot_general(qb, k_imag, (((1,), (0,)), ((), ())),
                                       precision=_MATCH))
        ir = inner[:bsz, :]
        ii = inner[bsz:, :]
        amp = (ir * ir + ii * ii) * occ_ref[...]
        col = i * tile + jax.lax.broadcasted_iota(jnp.int32, (1, tile), 1)
        amp = jnp.where(col < n_slots, amp, -jnp.inf)
        amps_scr[:, pl.ds(i * tile, tile)] = amp

        m_old = m_ref[...]
        m_new = jnp.maximum(m_old, jnp.max(amp, axis=1, keepdims=True))
        p = jnp.exp(amp - m_new)
        s_ref[...] = (s_ref[...] * jnp.exp(m_old - m_new)
                      + jnp.sum(p, axis=1, keepdims=True))
        m_ref[...] = m_new

        @pl.when(i == last)
        def _prefetch_first_contents():
            c_start_full(0, 0)

    @pl.when(phase == 1)
    def _phase1():
        slot = jax.lax.rem(i, 2)
        nxt = jax.lax.rem(i + 1, 2)

        @pl.when(i + 1 < last)
        def _prefetch_full():
            c_start_full(i + 1, nxt)

        @pl.when(i + 1 == last)
        def _prefetch_tail():
            c_start_tail(nxt)

        col = i * tile + jax.lax.broadcasted_iota(jnp.int32, (1, tile), 1)
        amp = amps_scr[:, pl.ds(i * tile, tile)]
        p = jnp.exp(amp - m_ref[...]) / s_ref[...]
        p = jnp.where(col < n_slots, p, 0.0)
        attn_ref[...] = p

        @pl.when(i < last)
        def _wait_f():
            c_wait_full(slot)

        @pl.when(i == last)
        def _wait_t():
            c_wait_tail(slot)

        acc = jax.lax.dot_general(p, cbuf[slot], (((1,), (0,)), ((), ())),
                                  precision=_MATCH)

        @pl.when(i == 0)
        def _first():
            cont_ref[...] = acc

        @pl.when(i != 0)
        def _rest():
            cont_ref[...] += acc


def kernel(query, contents, mem_keys, occupancy, W, b):
    n_slots, key_dim, _ = mem_keys.shape
    mem_dim = contents.shape[1]
    bsz = query.shape[0]

    k_t = mem_keys.transpose(1, 2, 0)
    c_t = contents.transpose(0, 2, 1)
    occ2 = occupancy.reshape(1, n_slots)
    b2 = b.reshape(1, 2 * key_dim)

    tile = _TILE
    nb = pl.cdiv(n_slots, tile)
    n_pad = nb * tile

    attn, cont = pl.pallas_call(
        functools.partial(_body, n_slots, tile, nb),
        grid=(2, nb),
        in_specs=[
            pl.BlockSpec((bsz, key_dim), lambda p, i: (0, 0)),
            pl.BlockSpec((2 * key_dim, key_dim), lambda p, i: (0, 0)),
            pl.BlockSpec((1, 2 * key_dim), lambda p, i: (0, 0)),
            pl.BlockSpec((1, tile), lambda p, i: (0, i * (1 - p))),
            pl.BlockSpec((key_dim, 2, tile), lambda p, i: (0, 0, i * (1 - p))),
            pl.BlockSpec(memory_space=pl.ANY),
        ],
        out_specs=[
            pl.BlockSpec((bsz, tile), lambda p, i: (0, i * p)),
            pl.BlockSpec((bsz, mem_dim), lambda p, i: (0, 0)),
        ],
        out_shape=[
            jax.ShapeDtypeStruct((bsz, n_slots), jnp.float32),
            jax.ShapeDtypeStruct((bsz, mem_dim), jnp.float32),
        ],
        scratch_shapes=[
            pltpu.VMEM((4 * bsz, key_dim), jnp.float32),
            pltpu.VMEM((bsz, n_pad), jnp.float32),
            pltpu.VMEM((bsz, 1), jnp.float32),
            pltpu.VMEM((bsz, 1), jnp.float32),
            pltpu.VMEM((2, tile, mem_dim), jnp.float32),
            pltpu.SemaphoreType.DMA((2,)),
        ],
        compiler_params=pltpu.CompilerParams(
            vmem_limit_bytes=100 * 1024 * 1024,
        ),
    )(query, W, b2, occ2, k_t, c_t)

    return (cont, attn)


# speedup vs baseline: 9.2630x; 1.0059x over previous
import functools

import jax
import jax.numpy as jnp
from jax.experimental import pallas as pl
from jax.experimental.pallas import tpu as pltpu

_MATCH = jax.lax.Precision.DEFAULT
_TILE = 12288


def _body(n_slots, tile, nb, q_ref, w_ref, b_ref, occ_ref, k_ref, c_any,
          attn_ref, cont_ref, qq_ref, amps_scr, m_ref, s_ref, cbuf, csem):
    phase = pl.program_id(0)
    i = pl.program_id(1)
    bsz = q_ref.shape[0]
    kd = w_ref.shape[0] // 2
    tail = n_slots - (nb - 1) * tile
    last = nb - 1

    def c_start_full(idx, slot):
        pltpu.make_async_copy(
            c_any.at[pl.ds(idx * tile, tile), 0],
            cbuf.at[slot],
            csem.at[slot]).start()

    def c_start_tail(slot):
        pltpu.make_async_copy(
            c_any.at[pl.ds(last * tile, tail), 0],
            cbuf.at[slot, pl.ds(0, tail)],
            csem.at[slot]).start()

    def c_wait_full(slot):
        pltpu.make_async_copy(
            c_any.at[pl.ds(0, tile), 0],
            cbuf.at[slot],
            csem.at[slot]).wait()

    def c_wait_tail(slot):
        pltpu.make_async_copy(
            c_any.at[pl.ds(0, tail), 0],
            cbuf.at[slot, pl.ds(0, tail)],
            csem.at[slot]).wait()

    @pl.when((phase == 0) & (i == 0))
    def _prologue():
        q_enc = jax.lax.dot_general(q_ref[...], w_ref[...],
                                    (((1,), (1,)), ((), ())),
                                    precision=_MATCH) + b_ref[...]
        q_real = q_enc[:, :kd]
        q_imag = q_enc[:, kd:]
        qa = jnp.concatenate([q_real, -q_imag], axis=0)
        qb = jnp.concatenate([q_imag, q_real], axis=0)
        qq_ref[...] = jnp.concatenate([qa, qb], axis=0)
        m_ref[...] = jnp.full((bsz, 1), -jnp.inf, dtype=jnp.float32)
        s_ref[...] = jnp.zeros((bsz, 1), dtype=jnp.float32)

    @pl.when(phase == 0)
    def _phase0():
        k_real = k_ref[:, 0, :]
        k_imag = k_ref[:, 1, :]
        qa = qq_ref[:2 * bsz, :]
        qb = qq_ref[2 * bsz:, :]
        inner = (jax.lax.dot_general(qa, k_real, (((1,), (0,)), ((), ())),
                                     precision=_MATCH)
                 + jax.lax.dot_general(qb, k_imag, (((1,), (0,)), ((), ())),
                                       precision=_MATCH))
        ir = inner[:bsz, :]
        ii = inner[bsz:, :]
        amp = (ir * ir + ii * ii) * occ_ref[...]
        col = i * tile + jax.lax.broadcasted_iota(jnp.int32, (1, tile), 1)
        amp = jnp.where(col < n_slots, amp, -jnp.inf)
        amps_scr[:, pl.ds(i * tile, tile)] = amp

        m_old = m_ref[...]
        m_new = jnp.maximum(m_old, jnp.max(amp, axis=1, keepdims=True))
        p = jnp.exp(amp - m_new)
        s_ref[...] = (s_ref[...] * jnp.exp(m_old - m_new)
                      + jnp.sum(p, axis=1, keepdims=True))
        m_ref[...] = m_new

        @pl.when(i == last)
        def _prefetch_first_contents():
            c_start_full(0, 0)

    @pl.when(phase == 1)
    def _phase1():
        slot = jax.lax.rem(i, 2)
        nxt = jax.lax.rem(i + 1, 2)

        @pl.when(i + 1 < last)
        def _prefetch_full():
            c_start_full(i + 1, nxt)

        @pl.when(i + 1 == last)
        def _prefetch_tail():
            c_start_tail(nxt)

        col = i * tile + jax.lax.broadcasted_iota(jnp.int32, (1, tile), 1)
        amp = amps_scr[:, pl.ds(i * tile, tile)]
        p = jnp.exp(amp - m_ref[...]) / s_ref[...]
        p = jnp.where(col < n_slots, p, 0.0)
        attn_ref[...] = p

        @pl.when(i < last)
        def _wait_f():
            c_wait_full(slot)

        @pl.when(i == last)
        def _wait_t():
            c_wait_tail(slot)

        acc = jax.lax.dot_general(p, cbuf[slot], (((1,), (0,)), ((), ())),
                                  precision=_MATCH)

        @pl.when(i == 0)
        def _first():
            cont_ref[...] = acc

        @pl.when(i != 0)
        def _rest():
            cont_ref[...] += acc


def kernel(query, contents, mem_keys, occupancy, W, b):
    n_slots, key_dim, _ = mem_keys.shape
    mem_dim = contents.shape[1]
    bsz = query.shape[0]

    k_t = mem_keys.transpose(1, 2, 0)
    c_t = contents.transpose(0, 2, 1)
    occ2 = occupancy.reshape(1, n_slots)
    b2 = b.reshape(1, 2 * key_dim)

    tile = _TILE
    nb = pl.cdiv(n_slots, tile)
    n_pad = nb * tile

    attn, cont = pl.pallas_call(
        functools.partial(_body, n_slots, tile, nb),
        grid=(2, nb),
        in_specs=[
            pl.BlockSpec((bsz, key_dim), lambda p, i: (0, 0)),
            pl.BlockSpec((2 * key_dim, key_dim), lambda p, i: (0, 0)),
            pl.BlockSpec((1, 2 * key_dim), lambda p, i: (0, 0)),
            pl.BlockSpec((1, tile), lambda p, i: (0, i * (1 - p))),
            pl.BlockSpec((key_dim, 2, tile), lambda p, i: (0, 0, i * (1 - p))),
            pl.BlockSpec(memory_space=pl.ANY),
        ],
        out_specs=[
            pl.BlockSpec((bsz, tile), lambda p, i: (0, i * p)),
            pl.BlockSpec((bsz, mem_dim), lambda p, i: (0, 0)),
        ],
        out_shape=[
            jax.ShapeDtypeStruct((bsz, n_slots), jnp.float32),
            jax.ShapeDtypeStruct((bsz, mem_dim), jnp.float32),
        ],
        scratch_shapes=[
            pltpu.VMEM((4 * bsz, key_dim), jnp.float32),
            pltpu.VMEM((bsz, n_pad), jnp.float32),
            pltpu.VMEM((bsz, 1), jnp.float32),
            pltpu.VMEM((bsz, 1), jnp.float32),
            pltpu.VMEM((2, tile, mem_dim), jnp.float32),
            pltpu.SemaphoreType.DMA((2,)),
        ],
        compiler_params=pltpu.CompilerParams(
            vmem_limit_bytes=114 * 1024 * 1024,
        ),
    )(query, W, b2, occ2, k_t, c_t)

    return (cont, attn)


# speedup vs baseline: 9.6658x; 1.0435x over previous
import functools

import jax
import jax.numpy as jnp
from jax.experimental import pallas as pl
from jax.experimental.pallas import tpu as pltpu

_MATCH = jax.lax.Precision.DEFAULT
_TILE = 8192
_NBUF = 4


def _body(n_slots, tile, nb, q_ref, w_ref, b_ref, occ_ref, k_ref, c_any,
          attn_ref, cont_ref, qq_ref, amps_scr, m_ref, s_ref, cbuf, csem):
    phase = pl.program_id(0)
    i = pl.program_id(1)
    bsz = q_ref.shape[0]
    kd = w_ref.shape[0] // 2
    tail = n_slots - (nb - 1) * tile
    last = nb - 1

    def c_start_full(idx, slot):
        pltpu.make_async_copy(
            c_any.at[pl.ds(idx * tile, tile), 0],
            cbuf.at[slot],
            csem.at[slot]).start()

    def c_start_tail(slot):
        pltpu.make_async_copy(
            c_any.at[pl.ds(last * tile, tail), 0],
            cbuf.at[slot, pl.ds(0, tail)],
            csem.at[slot]).start()

    def c_wait_full(slot):
        pltpu.make_async_copy(
            c_any.at[pl.ds(0, tile), 0],
            cbuf.at[slot],
            csem.at[slot]).wait()

    def c_wait_tail(slot):
        pltpu.make_async_copy(
            c_any.at[pl.ds(0, tail), 0],
            cbuf.at[slot, pl.ds(0, tail)],
            csem.at[slot]).wait()

    @pl.when((phase == 0) & (i == 0))
    def _prologue():
        q_enc = jax.lax.dot_general(q_ref[...], w_ref[...],
                                    (((1,), (1,)), ((), ())),
                                    precision=_MATCH) + b_ref[...]
        q_real = q_enc[:, :kd]
        q_imag = q_enc[:, kd:]
        qa = jnp.concatenate([q_real, -q_imag], axis=0)
        qb = jnp.concatenate([q_imag, q_real], axis=0)
        qq_ref[...] = jnp.concatenate([qa, qb], axis=0)
        m_ref[...] = jnp.full((bsz, 1), -jnp.inf, dtype=jnp.float32)
        s_ref[...] = jnp.zeros((bsz, 1), dtype=jnp.float32)

    @pl.when(phase == 0)
    def _phase0():
        k_real = k_ref[:, 0, :]
        k_imag = k_ref[:, 1, :]
        qa = qq_ref[:2 * bsz, :]
        qb = qq_ref[2 * bsz:, :]
        inner = (jax.lax.dot_general(qa, k_real, (((1,), (0,)), ((), ())),
                                     precision=_MATCH)
                 + jax.lax.dot_general(qb, k_imag, (((1,), (0,)), ((), ())),
                                       precision=_MATCH))
        ir = inner[:bsz, :]
        ii = inner[bsz:, :]
        amp = (ir * ir + ii * ii) * occ_ref[...]
        col = i * tile + jax.lax.broadcasted_iota(jnp.int32, (1, tile), 1)
        amp = jnp.where(col < n_slots, amp, -jnp.inf)
        amps_scr[:, pl.ds(i * tile, tile)] = amp

        m_old = m_ref[...]
        m_new = jnp.maximum(m_old, jnp.max(amp, axis=1, keepdims=True))
        p = jnp.exp(amp - m_new)
        s_ref[...] = (s_ref[...] * jnp.exp(m_old - m_new)
                      + jnp.sum(p, axis=1, keepdims=True))
        m_ref[...] = m_new

        nbuf = cbuf.shape[0]
        ahead = nbuf - 1
        for k in range(ahead):
            @pl.when(i == last - (ahead - 1) + k)
            def _prefetch_contents(k=k):
                c_start_full(k, k)

    @pl.when(phase == 1)
    def _phase1():
        nbuf = cbuf.shape[0]
        ahead = nbuf - 1
        slot = jax.lax.rem(i, nbuf)
        nxt = jax.lax.rem(i + ahead, nbuf)

        @pl.when(i + ahead < last)
        def _prefetch_full():
            c_start_full(i + ahead, nxt)

        @pl.when(i + ahead == last)
        def _prefetch_tail():
            c_start_tail(nxt)

        col = i * tile + jax.lax.broadcasted_iota(jnp.int32, (1, tile), 1)
        amp = amps_scr[:, pl.ds(i * tile, tile)]
        rs = 1.0 / s_ref[...]
        p = jnp.exp(amp - m_ref[...]) * rs
        p = jnp.where(col < n_slots, p, 0.0)
        attn_ref[...] = p

        @pl.when(i < last)
        def _wait_f():
            c_wait_full(slot)

        @pl.when(i == last)
        def _wait_t():
            c_wait_tail(slot)

        acc = jax.lax.dot_general(p, cbuf[slot], (((1,), (0,)), ((), ())),
                                  precision=_MATCH)

        @pl.when(i == 0)
        def _first():
            cont_ref[...] = acc

        @pl.when(i != 0)
        def _rest():
            cont_ref[...] += acc


def kernel(query, contents, mem_keys, occupancy, W, b):
    n_slots, key_dim, _ = mem_keys.shape
    mem_dim = contents.shape[1]
    bsz = query.shape[0]

    k_t = mem_keys.transpose(1, 2, 0)
    c_t = contents.transpose(0, 2, 1)
    occ2 = occupancy.reshape(1, n_slots)
    b2 = b.reshape(1, 2 * key_dim)

    tile = _TILE
    nb = pl.cdiv(n_slots, tile)
    n_pad = nb * tile

    attn, cont = pl.pallas_call(
        functools.partial(_body, n_slots, tile, nb),
        grid=(2, nb),
        in_specs=[
            pl.BlockSpec((bsz, key_dim), lambda p, i: (0, 0)),
            pl.BlockSpec((2 * key_dim, key_dim), lambda p, i: (0, 0)),
            pl.BlockSpec((1, 2 * key_dim), lambda p, i: (0, 0)),
            pl.BlockSpec((1, tile), lambda p, i: (0, i * (1 - p))),
            pl.BlockSpec((key_dim, 2, tile), lambda p, i: (0, 0, i * (1 - p))),
            pl.BlockSpec(memory_space=pl.ANY),
        ],
        out_specs=[
            pl.BlockSpec((bsz, tile), lambda p, i: (0, i * p)),
            pl.BlockSpec((bsz, mem_dim), lambda p, i: (0, 0)),
        ],
        out_shape=[
            jax.ShapeDtypeStruct((bsz, n_slots), jnp.float32),
            jax.ShapeDtypeStruct((bsz, mem_dim), jnp.float32),
        ],
        scratch_shapes=[
            pltpu.VMEM((4 * bsz, key_dim), jnp.float32),
            pltpu.VMEM((bsz, n_pad), jnp.float32),
            pltpu.VMEM((bsz, 1), jnp.float32),
            pltpu.VMEM((bsz, 1), jnp.float32),
            pltpu.VMEM((_NBUF, tile, mem_dim), jnp.float32),
            pltpu.SemaphoreType.DMA((_NBUF,)),
        ],
        compiler_params=pltpu.CompilerParams(
            vmem_limit_bytes=114 * 1024 * 1024,
        ),
    )(query, W, b2, occ2, k_t, c_t)

    return (cont, attn)


# speedup vs baseline: 9.8309x; 1.0171x over previous
import functools

import jax
import jax.numpy as jnp
from jax.experimental import pallas as pl
from jax.experimental.pallas import tpu as pltpu

_MATCH = jax.lax.Precision.DEFAULT
_TILE = 8192
_NBUF = 5


def _body(n_slots, tile, nb, q_ref, w_ref, b_ref, occ_ref, k_ref, c_any,
          attn_ref, cont_ref, qq_ref, amps_scr, m_ref, s_ref, cbuf, csem):
    phase = pl.program_id(0)
    i = pl.program_id(1)
    bsz = q_ref.shape[0]
    kd = w_ref.shape[0] // 2
    tail = n_slots - (nb - 1) * tile
    last = nb - 1

    def c_start_full(idx, slot):
        pltpu.make_async_copy(
            c_any.at[pl.ds(idx * tile, tile), 0],
            cbuf.at[slot],
            csem.at[slot]).start()

    def c_start_tail(slot):
        pltpu.make_async_copy(
            c_any.at[pl.ds(last * tile, tail), 0],
            cbuf.at[slot, pl.ds(0, tail)],
            csem.at[slot]).start()

    def c_wait_full(slot):
        pltpu.make_async_copy(
            c_any.at[pl.ds(0, tile), 0],
            cbuf.at[slot],
            csem.at[slot]).wait()

    def c_wait_tail(slot):
        pltpu.make_async_copy(
            c_any.at[pl.ds(0, tail), 0],
            cbuf.at[slot, pl.ds(0, tail)],
            csem.at[slot]).wait()

    @pl.when((phase == 0) & (i == 0))
    def _prologue():
        q_enc = jax.lax.dot_general(q_ref[...], w_ref[...],
                                    (((1,), (1,)), ((), ())),
                                    precision=_MATCH) + b_ref[...]
        q_real = q_enc[:, :kd]
        q_imag = q_enc[:, kd:]
        qa = jnp.concatenate([q_real, -q_imag], axis=0)
        qb = jnp.concatenate([q_imag, q_real], axis=0)
        qq_ref[...] = jnp.concatenate([qa, qb], axis=0)
        m_ref[...] = jnp.full((bsz, 1), -jnp.inf, dtype=jnp.float32)
        s_ref[...] = jnp.zeros((bsz, 1), dtype=jnp.float32)

    @pl.when(phase == 0)
    def _phase0():
        k_real = k_ref[:, 0, :]
        k_imag = k_ref[:, 1, :]
        qa = qq_ref[:2 * bsz, :]
        qb = qq_ref[2 * bsz:, :]
        inner = (jax.lax.dot_general(qa, k_real, (((1,), (0,)), ((), ())),
                                     precision=_MATCH)
                 + jax.lax.dot_general(qb, k_imag, (((1,), (0,)), ((), ())),
                                       precision=_MATCH))
        ir = inner[:bsz, :]
        ii = inner[bsz:, :]
        amp = (ir * ir + ii * ii) * occ_ref[...]
        col = i * tile + jax.lax.broadcasted_iota(jnp.int32, (1, tile), 1)
        amp = jnp.where(col < n_slots, amp, -jnp.inf)
        amps_scr[:, pl.ds(i * tile, tile)] = amp

        m_old = m_ref[...]
        m_new = jnp.maximum(m_old, jnp.max(amp, axis=1, keepdims=True))
        p = jnp.exp(amp - m_new)
        s_ref[...] = (s_ref[...] * jnp.exp(m_old - m_new)
                      + jnp.sum(p, axis=1, keepdims=True))
        m_ref[...] = m_new

        nbuf = cbuf.shape[0]
        ahead = nbuf - 1
        for k in range(ahead):
            @pl.when(i == last - (ahead - 1) + k)
            def _prefetch_contents(k=k):
                c_start_full(k, k)

    @pl.when(phase == 1)
    def _phase1():
        nbuf = cbuf.shape[0]
        ahead = nbuf - 1
        slot = jax.lax.rem(i, nbuf)
        nxt = jax.lax.rem(i + ahead, nbuf)

        @pl.when(i + ahead < last)
        def _prefetch_full():
            c_start_full(i + ahead, nxt)

        @pl.when(i + ahead == last)
        def _prefetch_tail():
            c_start_tail(nxt)

        col = i * tile + jax.lax.broadcasted_iota(jnp.int32, (1, tile), 1)
        amp = amps_scr[:, pl.ds(i * tile, tile)]
        rs = 1.0 / s_ref[...]
        p = jnp.exp(amp - m_ref[...]) * rs
        p = jnp.where(col < n_slots, p, 0.0)
        attn_ref[...] = p

        @pl.when(i < last)
        def _wait_f():
            c_wait_full(slot)

        @pl.when(i == last)
        def _wait_t():
            c_wait_tail(slot)

        acc = jax.lax.dot_general(p, cbuf[slot], (((1,), (0,)), ((), ())),
                                  precision=_MATCH)

        @pl.when(i == 0)
        def _first():
            cont_ref[...] = acc

        @pl.when(i != 0)
        def _rest():
            cont_ref[...] += acc


def kernel(query, contents, mem_keys, occupancy, W, b):
    n_slots, key_dim, _ = mem_keys.shape
    mem_dim = contents.shape[1]
    bsz = query.shape[0]

    k_t = mem_keys.transpose(1, 2, 0)
    c_t = contents.transpose(0, 2, 1)
    occ2 = occupancy.reshape(1, n_slots)
    b2 = b.reshape(1, 2 * key_dim)

    tile = _TILE
    nb = pl.cdiv(n_slots, tile)
    n_pad = nb * tile

    attn, cont = pl.pallas_call(
        functools.partial(_body, n_slots, tile, nb),
        grid=(2, nb),
        in_specs=[
            pl.BlockSpec((bsz, key_dim), lambda p, i: (0, 0)),
            pl.BlockSpec((2 * key_dim, key_dim), lambda p, i: (0, 0)),
            pl.BlockSpec((1, 2 * key_dim), lambda p, i: (0, 0)),
            pl.BlockSpec((1, tile), lambda p, i: (0, i * (1 - p))),
            pl.BlockSpec((key_dim, 2, tile), lambda p, i: (0, 0, i * (1 - p))),
            pl.BlockSpec(memory_space=pl.ANY),
        ],
        out_specs=[
            pl.BlockSpec((bsz, tile), lambda p, i: (0, i * p)),
            pl.BlockSpec((bsz, mem_dim), lambda p, i: (0, 0)),
        ],
        out_shape=[
            jax.ShapeDtypeStruct((bsz, n_slots), jnp.float32),
            jax.ShapeDtypeStruct((bsz, mem_dim), jnp.float32),
        ],
        scratch_shapes=[
            pltpu.VMEM((4 * bsz, key_dim), jnp.float32),
            pltpu.VMEM((bsz, n_pad), jnp.float32),
            pltpu.VMEM((bsz, 1), jnp.float32),
            pltpu.VMEM((bsz, 1), jnp.float32),
            pltpu.VMEM((_NBUF, tile, mem_dim), jnp.float32),
            pltpu.SemaphoreType.DMA((_NBUF,)),
        ],
        compiler_params=pltpu.CompilerParams(
            vmem_limit_bytes=114 * 1024 * 1024,
        ),
    )(query, W, b2, occ2, k_t, c_t)

    return (cont, attn)
